# baseline (speedup 1.0000x reference)
import jax
import jax.numpy as jnp
from jax.experimental import pallas as pl
from jax.experimental.pallas import tpu as pltpu

N = 4096
EMBED = 64
HEAD_NUM = 4
D_H = EMBED // HEAD_NUM
MODEL_CAT_RATE = 0.02
ID_CAT_RATE = 0.36
IMG_DIM = 4096
TXT_DIM = 1024

_F32 = jnp.float32


def _dot(a, b):
    return jax.lax.dot_general(a, b, (((1,), (0,)), ((), ())),
                               preferred_element_type=_F32)


def _rownorm(x):
    n = jnp.sqrt(jnp.sum(x * x, axis=1, keepdims=True))
    return x / jnp.maximum(n, 1e-12)


def _mhsa_mean(a, b, w_q, w_k, w_cat):
    qa = _dot(a, w_q)
    qb = _dot(b, w_q)
    ka = _dot(a, w_k)
    kb = _dot(b, w_k)
    scale = 1.0 / jnp.sqrt(jnp.float32(D_H))
    z_parts_a = []
    z_parts_b = []
    for h in range(HEAD_NUM):
        s = slice(h * D_H, (h + 1) * D_H)
        qah, qbh = qa[:, s], qb[:, s]
        kah, kbh = ka[:, s], kb[:, s]
        l_aa = jnp.sum(qah * kah, axis=1, keepdims=True) * scale
        l_ab = jnp.sum(qah * kbh, axis=1, keepdims=True) * scale
        l_ba = jnp.sum(qbh * kah, axis=1, keepdims=True) * scale
        l_bb = jnp.sum(qbh * kbh, axis=1, keepdims=True) * scale
        m_a = jnp.maximum(l_aa, l_ab)
        e_aa = jnp.exp(l_aa - m_a)
        e_ab = jnp.exp(l_ab - m_a)
        za = (e_aa * a + e_ab * b) / (e_aa + e_ab)
        m_b = jnp.maximum(l_ba, l_bb)
        e_ba = jnp.exp(l_ba - m_b)
        e_bb = jnp.exp(l_bb - m_b)
        zb = (e_ba * a + e_bb * b) / (e_ba + e_bb)
        z_parts_a.append(za)
        z_parts_b.append(zb)
    zcat_a = jnp.concatenate(z_parts_a, axis=1)
    zcat_b = jnp.concatenate(z_parts_b, axis=1)
    out_a = _dot(zcat_a, w_cat)
    out_b = _dot(zcat_b, w_cat)
    return 0.5 * (out_a + out_b)


def _stage_a_kernel(img_ui_ref, txt_ui_ref, img_iu_ref, txt_iu_ref,
                    img_raw_ref, txt_raw_ref,
                    w_img_ref, b_img_ref, w_txt_ref, b_txt_ref,
                    uemb_full_ref, iemb_full_ref,
                    uemb_blk_ref, iemb_blk_ref,
                    w_q_ref, w_k_ref, w_cat_ref,
                    img_feats_ref, txt_feats_ref,
                    iu_id_ref, tu_id_ref, ug0_ref, ig0_ref):
    img_feats_ref[...] = _dot(img_raw_ref[...], w_img_ref[...]) + b_img_ref[...]
    txt_feats_ref[...] = _dot(txt_raw_ref[...], w_txt_ref[...]) + b_txt_ref[...]
    iemb_full = iemb_full_ref[...]
    uemb_full = uemb_full_ref[...]
    iu_id = _dot(img_ui_ref[...], iemb_full)
    tu_id = _dot(txt_ui_ref[...], iemb_full)
    ii_id = _dot(img_iu_ref[...], uemb_full)
    ti_id = _dot(txt_iu_ref[...], uemb_full)
    iu_id_ref[...] = iu_id
    tu_id_ref[...] = tu_id
    w_q, w_k, w_cat = w_q_ref[...], w_k_ref[...], w_cat_ref[...]
    user_emb = _mhsa_mean(iu_id, tu_id, w_q, w_k, w_cat)
    item_emb = _mhsa_mean(ii_id, ti_id, w_q, w_k, w_cat)
    ug0_ref[...] = uemb_blk_ref[...] + ID_CAT_RATE * _rownorm(user_emb)
    ig0_ref[...] = iemb_blk_ref[...] + ID_CAT_RATE * _rownorm(item_emb)


RM = 256
NBM = N // RM


def _mega_kernel(ui_ref, iu_ref, rhs_ref, ug0_ref, ig0_ref,
                 iuf_ref, tuf_ref, iif_ref, tif_ref, ufin_ref, ifin_ref,
                 ui16_s, cpack_s, dpack_s, ug2_s, acc_s):
    p = pl.program_id(0)
    b = pl.program_id(1)
    r0 = b * RM

    @pl.when(p == 0)
    def _phase_c():
        g16 = ui_ref[...].astype(jnp.bfloat16)
        ui16_s[pl.ds(r0, RM), :] = g16
        cp = _dot(g16, rhs_ref[...])
        cpack_s[pl.ds(r0, RM), :] = cp.astype(jnp.bfloat16)
        iuf_ref[...] = cp[:, 0:EMBED]
        tuf_ref[...] = cp[:, EMBED:2 * EMBED]

    @pl.when(p == 1)
    def _phase_d():
        g16 = iu_ref[...].astype(jnp.bfloat16)
        dp = _dot(g16, cpack_s[...])
        dpack_s[pl.ds(r0, RM), :] = dp.astype(jnp.bfloat16)
        iif_ref[...] = dp[:, 0:EMBED]
        tif_ref[...] = dp[:, EMBED:2 * EMBED]
        contrib = _dot(ui16_s[:, pl.ds(r0, RM)],
                       dp[:, 2 * EMBED:3 * EMBED].astype(jnp.bfloat16))

        @pl.when(b == 0)
        def _():
            acc_s[...] = contrib

        @pl.when(b > 0)
        def _():
            acc_s[...] = acc_s[...] + contrib

        @pl.when(b == NBM - 1)
        def _():
            ug2 = jax.nn.softmax(acc_s[...], axis=-1)
            ug2_s[...] = ug2.astype(jnp.bfloat16)
            acc_s[...] = ug2

    @pl.when(p == 2)
    def _phase_f():
        g16 = iu_ref[...].astype(jnp.bfloat16)
        t = _dot(g16, ug2_s[...])
        ig2 = jax.nn.softmax(t, axis=-1)
        dp = dpack_s[pl.ds(r0, RM), :].astype(_F32)
        iif = dp[:, 0:EMBED]
        tif = dp[:, EMBED:2 * EMBED]
        ig1 = dp[:, 2 * EMBED:3 * EMBED]
        i = (ig0_ref[...] + ig1 + ig2) * (1.0 / 3.0)
        ifin_ref[...] = (i + MODEL_CAT_RATE * _rownorm(iif)
                         + MODEL_CAT_RATE * _rownorm(tif))
        cp = cpack_s[pl.ds(r0, RM), :].astype(_F32)
        iuf = cp[:, 0:EMBED]
        tuf = cp[:, EMBED:2 * EMBED]
        ug1 = cp[:, 2 * EMBED:3 * EMBED]
        ug2 = acc_s[pl.ds(r0, RM), :]
        u = (ug0_ref[...] + ug1 + ug2) * (1.0 / 3.0)
        ufin_ref[...] = (u + MODEL_CAT_RATE * _rownorm(iuf)
                         + MODEL_CAT_RATE * _rownorm(tuf))


def _row_spec(r, cols):
    return pl.BlockSpec((r, cols), lambda b: (b, 0))


def _full_spec(rows, cols):
    return pl.BlockSpec((rows, cols), lambda b: (0, 0))


_ARB = pltpu.CompilerParams(dimension_semantics=("arbitrary",))


def kernel(ui_graph, iu_graph, image_ui_graph, image_iu_graph, text_ui_graph,
           text_iu_graph, image_feats_raw, text_feats_raw, W_img, b_img,
           W_txt, b_txt, user_id_emb, item_id_emb, w_q, w_k, w_cat):
    f32 = _F32
    b_img2 = b_img.reshape(1, EMBED)
    b_txt2 = b_txt.reshape(1, EMBED)

    RA = 256
    (image_feats, text_feats, image_user_id, text_user_id,
     u_g0, i_g0) = pl.pallas_call(
        _stage_a_kernel,
        grid=(N // RA,),
        in_specs=[
            _row_spec(RA, N),
            _row_spec(RA, N),
            _row_spec(RA, N),
            _row_spec(RA, N),
            _row_spec(RA, IMG_DIM),
            _row_spec(RA, TXT_DIM),
            _full_spec(IMG_DIM, EMBED),
            _full_spec(1, EMBED),
            _full_spec(TXT_DIM, EMBED),
            _full_spec(1, EMBED),
            _full_spec(N, EMBED),
            _full_spec(N, EMBED),
            _row_spec(RA, EMBED),
            _row_spec(RA, EMBED),
            _full_spec(EMBED, EMBED),
            _full_spec(EMBED, EMBED),
            _full_spec(HEAD_NUM * EMBED, EMBED),
        ],
        out_specs=[_row_spec(RA, EMBED)] * 6,
        out_shape=[jax.ShapeDtypeStruct((N, EMBED), f32)] * 6,
        compiler_params=_ARB,
    )(image_ui_graph, text_ui_graph, image_iu_graph, text_iu_graph,
      image_feats_raw, text_feats_raw, W_img, b_img2, W_txt, b_txt2,
      user_id_emb, item_id_emb, user_id_emb, item_id_emb, w_q, w_k, w_cat)

    rhs_c = jnp.concatenate([image_feats, text_feats, i_g0],
                            axis=1).astype(jnp.bfloat16)
    last = NBM - 1

    (image_user_feats, text_user_feats, image_item_feats, text_item_feats,
     u_g, i_g) = pl.pallas_call(
        _mega_kernel,
        grid=(3, NBM),
        in_specs=[
            pl.BlockSpec((RM, N),
                         lambda p, b: (jnp.where(p == 0, b, last), 0)),
            pl.BlockSpec((RM, N),
                         lambda p, b: (jnp.where(p >= 1, b, last), 0)),
            pl.BlockSpec((N, 3 * EMBED), lambda p, b: (0, 0)),
            pl.BlockSpec((RM, EMBED),
                         lambda p, b: (jnp.where(p == 2, b, 0), 0)),
            pl.BlockSpec((RM, EMBED),
                         lambda p, b: (jnp.where(p == 2, b, 0), 0)),
        ],
        out_specs=[
            pl.BlockSpec((RM, EMBED),
                         lambda p, b: (jnp.where(p == 0, b, last), 0)),
            pl.BlockSpec((RM, EMBED),
                         lambda p, b: (jnp.where(p == 0, b, last), 0)),
            pl.BlockSpec((RM, EMBED),
                         lambda p, b: (jnp.where(p == 1, b,
                                       jnp.where(p < 1, 0, last)), 0)),
            pl.BlockSpec((RM, EMBED),
                         lambda p, b: (jnp.where(p == 1, b,
                                       jnp.where(p < 1, 0, last)), 0)),
            pl.BlockSpec((RM, EMBED),
                         lambda p, b: (jnp.where(p == 2, b, 0), 0)),
            pl.BlockSpec((RM, EMBED),
                         lambda p, b: (jnp.where(p == 2, b, 0), 0)),
        ],
        out_shape=[jax.ShapeDtypeStruct((N, EMBED), f32)] * 6,
        scratch_shapes=[
            pltpu.VMEM((N, N), jnp.bfloat16),
            pltpu.VMEM((N, 3 * EMBED), jnp.bfloat16),
            pltpu.VMEM((N, 3 * EMBED), jnp.bfloat16),
            pltpu.VMEM((N, EMBED), jnp.bfloat16),
            pltpu.VMEM((N, EMBED), f32),
        ],
        compiler_params=pltpu.CompilerParams(
            dimension_semantics=("arbitrary", "arbitrary"),
            vmem_limit_bytes=64 * 1024 * 1024),
    )(ui_graph, iu_graph, rhs_c, u_g0, i_g0)

    return (u_g, i_g, image_item_feats, text_item_feats, image_user_feats,
            text_user_feats, u_g, i_g, image_user_id, text_user_id)


# speedup vs baseline: 1.0080x; 1.0080x over previous
import jax
import jax.numpy as jnp
from jax.experimental import pallas as pl
from jax.experimental.pallas import tpu as pltpu

N = 4096
EMBED = 64
HEAD_NUM = 4
D_H = EMBED // HEAD_NUM
MODEL_CAT_RATE = 0.02
ID_CAT_RATE = 0.36
IMG_DIM = 4096
TXT_DIM = 1024

_F32 = jnp.float32


def _dot(a, b):
    return jax.lax.dot_general(a, b, (((1,), (0,)), ((), ())),
                               preferred_element_type=_F32)


def _rownorm(x):
    n = jnp.sqrt(jnp.sum(x * x, axis=1, keepdims=True))
    return x / jnp.maximum(n, 1e-12)


def _mhsa_mean(a, b, w_q, w_k, w_cat):
    qa = _dot(a, w_q)
    qb = _dot(b, w_q)
    ka = _dot(a, w_k)
    kb = _dot(b, w_k)
    scale = 1.0 / jnp.sqrt(jnp.float32(D_H))
    z_parts_a = []
    z_parts_b = []
    for h in range(HEAD_NUM):
        s = slice(h * D_H, (h + 1) * D_H)
        qah, qbh = qa[:, s], qb[:, s]
        kah, kbh = ka[:, s], kb[:, s]
        l_aa = jnp.sum(qah * kah, axis=1, keepdims=True) * scale
        l_ab = jnp.sum(qah * kbh, axis=1, keepdims=True) * scale
        l_ba = jnp.sum(qbh * kah, axis=1, keepdims=True) * scale
        l_bb = jnp.sum(qbh * kbh, axis=1, keepdims=True) * scale
        m_a = jnp.maximum(l_aa, l_ab)
        e_aa = jnp.exp(l_aa - m_a)
        e_ab = jnp.exp(l_ab - m_a)
        za = (e_aa * a + e_ab * b) / (e_aa + e_ab)
        m_b = jnp.maximum(l_ba, l_bb)
        e_ba = jnp.exp(l_ba - m_b)
        e_bb = jnp.exp(l_bb - m_b)
        zb = (e_ba * a + e_bb * b) / (e_ba + e_bb)
        z_parts_a.append(za)
        z_parts_b.append(zb)
    zcat_a = jnp.concatenate(z_parts_a, axis=1)
    zcat_b = jnp.concatenate(z_parts_b, axis=1)
    out_a = _dot(zcat_a, w_cat)
    out_b = _dot(zcat_b, w_cat)
    return 0.5 * (out_a + out_b)


def _stage_a_kernel(img_ui_ref, txt_ui_ref, img_iu_ref, txt_iu_ref,
                    img_raw_ref, txt_raw_ref,
                    w_img_ref, b_img_ref, w_txt_ref, b_txt_ref,
                    uemb_full_ref, iemb_full_ref,
                    uemb_blk_ref, iemb_blk_ref,
                    w_q_ref, w_k_ref, w_cat_ref,
                    img_feats_ref, txt_feats_ref,
                    iu_id_ref, tu_id_ref, ug0_ref, ig0_ref):
    img_feats_ref[...] = _dot(img_raw_ref[...], w_img_ref[...]) + b_img_ref[...]
    txt_feats_ref[...] = _dot(txt_raw_ref[...], w_txt_ref[...]) + b_txt_ref[...]
    iemb_full = iemb_full_ref[...]
    uemb_full = uemb_full_ref[...]
    iu_id = _dot(img_ui_ref[...], iemb_full)
    tu_id = _dot(txt_ui_ref[...], iemb_full)
    ii_id = _dot(img_iu_ref[...], uemb_full)
    ti_id = _dot(txt_iu_ref[...], uemb_full)
    iu_id_ref[...] = iu_id
    tu_id_ref[...] = tu_id
    w_q, w_k, w_cat = w_q_ref[...], w_k_ref[...], w_cat_ref[...]
    user_emb = _mhsa_mean(iu_id, tu_id, w_q, w_k, w_cat)
    item_emb = _mhsa_mean(ii_id, ti_id, w_q, w_k, w_cat)
    ug0_ref[...] = uemb_blk_ref[...] + ID_CAT_RATE * _rownorm(user_emb)
    ig0_ref[...] = iemb_blk_ref[...] + ID_CAT_RATE * _rownorm(item_emb)


RM = 256
NBM = N // RM


def _mega_kernel(ui_ref, iu_ref, rhs_ref, ug0_ref, ig0_ref,
                 iuf_ref, tuf_ref, iif_ref, tif_ref, ufin_ref, ifin_ref,
                 ui16_s, cpack_s, dpack_s, ug2_s):
    p = pl.program_id(0)
    b = pl.program_id(1)
    r0 = b * RM

    @pl.when(p == 0)
    def _phase_c():
        g16 = ui_ref[...].astype(jnp.bfloat16)
        ui16_s[pl.ds(r0, RM), :] = g16
        cp = _dot(g16, rhs_ref[...])
        cpack_s[pl.ds(r0, RM), :] = cp.astype(jnp.bfloat16)
        iuf_ref[...] = cp[:, 0:EMBED]
        tuf_ref[...] = cp[:, EMBED:2 * EMBED]

    @pl.when(p == 1)
    def _phase_d():
        g16 = iu_ref[...].astype(jnp.bfloat16)
        dp = _dot(g16, cpack_s[...])
        dpack_s[pl.ds(r0, RM), :] = dp.astype(jnp.bfloat16)
        iif_ref[...] = dp[:, 0:EMBED]
        tif_ref[...] = dp[:, EMBED:2 * EMBED]

    @pl.when(p == 2)
    def _phase_e():
        g16 = ui16_s[pl.ds(r0, RM), :]
        t = _dot(g16, dpack_s[:, 2 * EMBED:3 * EMBED])
        ug2 = jax.nn.softmax(t, axis=-1)
        ug2_s[pl.ds(r0, RM), :] = ug2.astype(jnp.bfloat16)
        cp = cpack_s[pl.ds(r0, RM), :].astype(_F32)
        iuf = cp[:, 0:EMBED]
        tuf = cp[:, EMBED:2 * EMBED]
        ug1 = cp[:, 2 * EMBED:3 * EMBED]
        u = (ug0_ref[...] + ug1 + ug2) * (1.0 / 3.0)
        ufin_ref[...] = (u + MODEL_CAT_RATE * _rownorm(iuf)
                         + MODEL_CAT_RATE * _rownorm(tuf))

    @pl.when(p == 3)
    def _phase_f():
        g16 = iu_ref[...].astype(jnp.bfloat16)
        t = _dot(g16, ug2_s[...])
        ig2 = jax.nn.softmax(t, axis=-1)
        dp = dpack_s[pl.ds(r0, RM), :].astype(_F32)
        iif = dp[:, 0:EMBED]
        tif = dp[:, EMBED:2 * EMBED]
        ig1 = dp[:, 2 * EMBED:3 * EMBED]
        i = (ig0_ref[...] + ig1 + ig2) * (1.0 / 3.0)
        ifin_ref[...] = (i + MODEL_CAT_RATE * _rownorm(iif)
                         + MODEL_CAT_RATE * _rownorm(tif))


def _row_spec(r, cols):
    return pl.BlockSpec((r, cols), lambda b: (b, 0))


def _full_spec(rows, cols):
    return pl.BlockSpec((rows, cols), lambda b: (0, 0))


_ARB = pltpu.CompilerParams(dimension_semantics=("arbitrary",))


def kernel(ui_graph, iu_graph, image_ui_graph, image_iu_graph, text_ui_graph,
           text_iu_graph, image_feats_raw, text_feats_raw, W_img, b_img,
           W_txt, b_txt, user_id_emb, item_id_emb, w_q, w_k, w_cat):
    f32 = _F32
    b_img2 = b_img.reshape(1, EMBED)
    b_txt2 = b_txt.reshape(1, EMBED)

    RA = 256
    (image_feats, text_feats, image_user_id, text_user_id,
     u_g0, i_g0) = pl.pallas_call(
        _stage_a_kernel,
        grid=(N // RA,),
        in_specs=[
            _row_spec(RA, N),
            _row_spec(RA, N),
            _row_spec(RA, N),
            _row_spec(RA, N),
            _row_spec(RA, IMG_DIM),
            _row_spec(RA, TXT_DIM),
            _full_spec(IMG_DIM, EMBED),
            _full_spec(1, EMBED),
            _full_spec(TXT_DIM, EMBED),
            _full_spec(1, EMBED),
            _full_spec(N, EMBED),
            _full_spec(N, EMBED),
            _row_spec(RA, EMBED),
            _row_spec(RA, EMBED),
            _full_spec(EMBED, EMBED),
            _full_spec(EMBED, EMBED),
            _full_spec(HEAD_NUM * EMBED, EMBED),
        ],
        out_specs=[_row_spec(RA, EMBED)] * 6,
        out_shape=[jax.ShapeDtypeStruct((N, EMBED), f32)] * 6,
        compiler_params=_ARB,
    )(image_ui_graph, text_ui_graph, image_iu_graph, text_iu_graph,
      image_feats_raw, text_feats_raw, W_img, b_img2, W_txt, b_txt2,
      user_id_emb, item_id_emb, user_id_emb, item_id_emb, w_q, w_k, w_cat)

    rhs_c = jnp.concatenate([image_feats, text_feats, i_g0],
                            axis=1).astype(jnp.bfloat16)
    last = NBM - 1

    (image_user_feats, text_user_feats, image_item_feats, text_item_feats,
     u_g, i_g) = pl.pallas_call(
        _mega_kernel,
        grid=(4, NBM),
        in_specs=[
            pl.BlockSpec((RM, N),
                         lambda p, b: (jnp.where(p == 0, b, last), 0)),
            pl.BlockSpec((RM, N),
                         lambda p, b: (jnp.where((p == 1) | (p == 3), b, last), 0)),
            pl.BlockSpec((N, 3 * EMBED), lambda p, b: (0, 0)),
            pl.BlockSpec((RM, EMBED),
                         lambda p, b: (jnp.where(p == 2, b, 0), 0)),
            pl.BlockSpec((RM, EMBED),
                         lambda p, b: (jnp.where(p == 3, b, 0), 0)),
        ],
        out_specs=[
            pl.BlockSpec((RM, EMBED),
                         lambda p, b: (jnp.where(p == 0, b, last), 0)),
            pl.BlockSpec((RM, EMBED),
                         lambda p, b: (jnp.where(p == 0, b, last), 0)),
            pl.BlockSpec((RM, EMBED),
                         lambda p, b: (jnp.where(p == 1, b,
                                       jnp.where(p < 1, 0, last)), 0)),
            pl.BlockSpec((RM, EMBED),
                         lambda p, b: (jnp.where(p == 1, b,
                                       jnp.where(p < 1, 0, last)), 0)),
            pl.BlockSpec((RM, EMBED),
                         lambda p, b: (jnp.where(p == 2, b,
                                       jnp.where(p < 2, 0, last)), 0)),
            pl.BlockSpec((RM, EMBED),
                         lambda p, b: (jnp.where(p == 3, b, 0), 0)),
        ],
        out_shape=[jax.ShapeDtypeStruct((N, EMBED), f32)] * 6,
        scratch_shapes=[
            pltpu.VMEM((N, N), jnp.bfloat16),
            pltpu.VMEM((N, 3 * EMBED), jnp.bfloat16),
            pltpu.VMEM((N, 3 * EMBED), jnp.bfloat16),
            pltpu.VMEM((N, EMBED), jnp.bfloat16),
        ],
        compiler_params=pltpu.CompilerParams(
            dimension_semantics=("arbitrary", "arbitrary"),
            vmem_limit_bytes=64 * 1024 * 1024),
    )(ui_graph, iu_graph, rhs_c, u_g0, i_g0)

    return (u_g, i_g, image_item_feats, text_item_feats, image_user_feats,
            text_user_feats, u_g, i_g, image_user_id, text_user_id)


# speedup vs baseline: 1.0168x; 1.0088x over previous
import jax
import jax.numpy as jnp
from jax.experimental import pallas as pl
from jax.experimental.pallas import tpu as pltpu

N = 4096
EMBED = 64
HEAD_NUM = 4
D_H = EMBED // HEAD_NUM
MODEL_CAT_RATE = 0.02
ID_CAT_RATE = 0.36
IMG_DIM = 4096
TXT_DIM = 1024

_F32 = jnp.float32


def _dot(a, b):
    return jax.lax.dot_general(a, b, (((1,), (0,)), ((), ())),
                               preferred_element_type=_F32)


def _rownorm(x):
    n = jnp.sqrt(jnp.sum(x * x, axis=1, keepdims=True))
    return x / jnp.maximum(n, 1e-12)


def _mhsa_mean(a, b, w_q, w_k, w_cat):
    qa = _dot(a, w_q)
    qb = _dot(b, w_q)
    ka = _dot(a, w_k)
    kb = _dot(b, w_k)
    scale = 1.0 / jnp.sqrt(jnp.float32(D_H))
    z_parts_a = []
    z_parts_b = []
    for h in range(HEAD_NUM):
        s = slice(h * D_H, (h + 1) * D_H)
        qah, qbh = qa[:, s], qb[:, s]
        kah, kbh = ka[:, s], kb[:, s]
        l_aa = jnp.sum(qah * kah, axis=1, keepdims=True) * scale
        l_ab = jnp.sum(qah * kbh, axis=1, keepdims=True) * scale
        l_ba = jnp.sum(qbh * kah, axis=1, keepdims=True) * scale
        l_bb = jnp.sum(qbh * kbh, axis=1, keepdims=True) * scale
        m_a = jnp.maximum(l_aa, l_ab)
        e_aa = jnp.exp(l_aa - m_a)
        e_ab = jnp.exp(l_ab - m_a)
        za = (e_aa * a + e_ab * b) / (e_aa + e_ab)
        m_b = jnp.maximum(l_ba, l_bb)
        e_ba = jnp.exp(l_ba - m_b)
        e_bb = jnp.exp(l_bb - m_b)
        zb = (e_ba * a + e_bb * b) / (e_ba + e_bb)
        z_parts_a.append(za)
        z_parts_b.append(zb)
    zcat_a = jnp.concatenate(z_parts_a, axis=1)
    zcat_b = jnp.concatenate(z_parts_b, axis=1)
    out_a = _dot(zcat_a, w_cat)
    out_b = _dot(zcat_b, w_cat)
    return 0.5 * (out_a + out_b)


def _stage_a_kernel(img_ui_ref, txt_ui_ref, img_iu_ref, txt_iu_ref,
                    img_raw_ref, txt_raw_ref,
                    w_img_ref, b_img_ref, w_txt_ref, b_txt_ref,
                    uemb_full_ref, iemb_full_ref,
                    uemb_blk_ref, iemb_blk_ref,
                    w_q_ref, w_k_ref, w_cat_ref,
                    img_feats_ref, txt_feats_ref,
                    iu_id_ref, tu_id_ref, ug0_ref, ig0_ref):
    img_feats_ref[...] = _dot(img_raw_ref[...], w_img_ref[...]) + b_img_ref[...]
    txt_feats_ref[...] = _dot(txt_raw_ref[...], w_txt_ref[...]) + b_txt_ref[...]
    iemb_full = iemb_full_ref[...]
    uemb_full = uemb_full_ref[...]
    iu_id = _dot(img_ui_ref[...], iemb_full)
    tu_id = _dot(txt_ui_ref[...], iemb_full)
    ii_id = _dot(img_iu_ref[...], uemb_full)
    ti_id = _dot(txt_iu_ref[...], uemb_full)
    iu_id_ref[...] = iu_id
    tu_id_ref[...] = tu_id
    w_q, w_k, w_cat = w_q_ref[...], w_k_ref[...], w_cat_ref[...]
    user_emb = _mhsa_mean(iu_id, tu_id, w_q, w_k, w_cat)
    item_emb = _mhsa_mean(ii_id, ti_id, w_q, w_k, w_cat)
    ug0_ref[...] = uemb_blk_ref[...] + ID_CAT_RATE * _rownorm(user_emb)
    ig0_ref[...] = iemb_blk_ref[...] + ID_CAT_RATE * _rownorm(item_emb)


RM = 256
NBM = N // RM
RE = 1024


def _mega_kernel(ui_ref, iu_ref, rhs_ref, ug0_ref, ig0_ref,
                 iuf_ref, tuf_ref, iif_ref, tif_ref, ufin_ref, ifin_ref,
                 ui16_s, cpack_s, dpack_s, ug2_s):
    p = pl.program_id(0)
    b = pl.program_id(1)
    r0 = b * RM

    @pl.when(p == 0)
    def _phase_c():
        g16 = ui_ref[...].astype(jnp.bfloat16)
        ui16_s[pl.ds(r0, RM), :] = g16
        cp = _dot(g16, rhs_ref[...])
        cpack_s[pl.ds(r0, RM), :] = cp.astype(jnp.bfloat16)
        iuf_ref[...] = cp[:, 0:EMBED]
        tuf_ref[...] = cp[:, EMBED:2 * EMBED]

    @pl.when(p == 1)
    def _phase_d():
        g16 = iu_ref[...].astype(jnp.bfloat16)
        dp = _dot(g16, cpack_s[...])
        dpack_s[pl.ds(r0, RM), :] = dp.astype(jnp.bfloat16)
        iif_ref[...] = dp[:, 0:EMBED]
        tif_ref[...] = dp[:, EMBED:2 * EMBED]

    @pl.when((p == 2) & (b < N // RE))
    def _phase_e():
        e0 = b * RE
        g16 = ui16_s[pl.ds(e0, RE), :]
        t = _dot(g16, dpack_s[:, 2 * EMBED:3 * EMBED])
        ug2 = jax.nn.softmax(t, axis=-1)
        ug2_s[pl.ds(e0, RE), :] = ug2.astype(jnp.bfloat16)
        cp = cpack_s[pl.ds(e0, RE), :].astype(_F32)
        iuf = cp[:, 0:EMBED]
        tuf = cp[:, EMBED:2 * EMBED]
        ug1 = cp[:, 2 * EMBED:3 * EMBED]
        u = (ug0_ref[pl.ds(e0, RE), :] + ug1 + ug2) * (1.0 / 3.0)
        ufin_ref[pl.ds(e0, RE), :] = (u + MODEL_CAT_RATE * _rownorm(iuf)
                                      + MODEL_CAT_RATE * _rownorm(tuf))

    @pl.when(p == 3)
    def _phase_f():
        g16 = iu_ref[...].astype(jnp.bfloat16)
        t = _dot(g16, ug2_s[...])
        ig2 = jax.nn.softmax(t, axis=-1)
        dp = dpack_s[pl.ds(r0, RM), :].astype(_F32)
        iif = dp[:, 0:EMBED]
        tif = dp[:, EMBED:2 * EMBED]
        ig1 = dp[:, 2 * EMBED:3 * EMBED]
        i = (ig0_ref[...] + ig1 + ig2) * (1.0 / 3.0)
        ifin_ref[...] = (i + MODEL_CAT_RATE * _rownorm(iif)
                         + MODEL_CAT_RATE * _rownorm(tif))


def _row_spec(r, cols):
    return pl.BlockSpec((r, cols), lambda b: (b, 0))


def _full_spec(rows, cols):
    return pl.BlockSpec((rows, cols), lambda b: (0, 0))


_ARB = pltpu.CompilerParams(dimension_semantics=("arbitrary",))


def kernel(ui_graph, iu_graph, image_ui_graph, image_iu_graph, text_ui_graph,
           text_iu_graph, image_feats_raw, text_feats_raw, W_img, b_img,
           W_txt, b_txt, user_id_emb, item_id_emb, w_q, w_k, w_cat):
    f32 = _F32
    b_img2 = b_img.reshape(1, EMBED)
    b_txt2 = b_txt.reshape(1, EMBED)

    RA = 256
    (image_feats, text_feats, image_user_id, text_user_id,
     u_g0, i_g0) = pl.pallas_call(
        _stage_a_kernel,
        grid=(N // RA,),
        in_specs=[
            _row_spec(RA, N),
            _row_spec(RA, N),
            _row_spec(RA, N),
            _row_spec(RA, N),
            _row_spec(RA, IMG_DIM),
            _row_spec(RA, TXT_DIM),
            _full_spec(IMG_DIM, EMBED),
            _full_spec(1, EMBED),
            _full_spec(TXT_DIM, EMBED),
            _full_spec(1, EMBED),
            _full_spec(N, EMBED),
            _full_spec(N, EMBED),
            _row_spec(RA, EMBED),
            _row_spec(RA, EMBED),
            _full_spec(EMBED, EMBED),
            _full_spec(EMBED, EMBED),
            _full_spec(HEAD_NUM * EMBED, EMBED),
        ],
        out_specs=[_row_spec(RA, EMBED)] * 6,
        out_shape=[jax.ShapeDtypeStruct((N, EMBED), f32)] * 6,
        compiler_params=_ARB,
    )(image_ui_graph, text_ui_graph, image_iu_graph, text_iu_graph,
      image_feats_raw, text_feats_raw, W_img, b_img2, W_txt, b_txt2,
      user_id_emb, item_id_emb, user_id_emb, item_id_emb, w_q, w_k, w_cat)

    rhs_c = jnp.concatenate([image_feats, text_feats, i_g0],
                            axis=1).astype(jnp.bfloat16)
    last = NBM - 1

    (image_user_feats, text_user_feats, image_item_feats, text_item_feats,
     u_g, i_g) = pl.pallas_call(
        _mega_kernel,
        grid=(4, NBM),
        in_specs=[
            pl.BlockSpec((RM, N),
                         lambda p, b: (jnp.where(p == 0, b, last), 0)),
            pl.BlockSpec((RM, N),
                         lambda p, b: (jnp.where((p == 1) | (p == 3), b, last), 0)),
            pl.BlockSpec((N, 3 * EMBED), lambda p, b: (0, 0)),
            pl.BlockSpec((N, EMBED), lambda p, b: (0, 0)),
            pl.BlockSpec((RM, EMBED),
                         lambda p, b: (jnp.where(p == 3, b, 0), 0)),
        ],
        out_specs=[
            pl.BlockSpec((RM, EMBED),
                         lambda p, b: (jnp.where(p == 0, b, last), 0)),
            pl.BlockSpec((RM, EMBED),
                         lambda p, b: (jnp.where(p == 0, b, last), 0)),
            pl.BlockSpec((RM, EMBED),
                         lambda p, b: (jnp.where(p == 1, b,
                                       jnp.where(p < 1, 0, last)), 0)),
            pl.BlockSpec((RM, EMBED),
                         lambda p, b: (jnp.where(p == 1, b,
                                       jnp.where(p < 1, 0, last)), 0)),
            pl.BlockSpec((N, EMBED), lambda p, b: (0, 0)),
            pl.BlockSpec((RM, EMBED),
                         lambda p, b: (jnp.where(p == 3, b, 0), 0)),
        ],
        out_shape=[jax.ShapeDtypeStruct((N, EMBED), f32)] * 6,
        scratch_shapes=[
            pltpu.VMEM((N, N), jnp.bfloat16),
            pltpu.VMEM((N, 3 * EMBED), jnp.bfloat16),
            pltpu.VMEM((N, 3 * EMBED), jnp.bfloat16),
            pltpu.VMEM((N, EMBED), jnp.bfloat16),
        ],
        compiler_params=pltpu.CompilerParams(
            dimension_semantics=("arbitrary", "arbitrary"),
            vmem_limit_bytes=64 * 1024 * 1024),
    )(ui_graph, iu_graph, rhs_c, u_g0, i_g0)

    return (u_g, i_g, image_item_feats, text_item_feats, image_user_feats,
            text_user_feats, u_g, i_g, image_user_id, text_user_id)


# speedup vs baseline: 1.0317x; 1.0146x over previous
import jax
import jax.numpy as jnp
from jax.experimental import pallas as pl
from jax.experimental.pallas import tpu as pltpu

N = 4096
EMBED = 64
HEAD_NUM = 4
D_H = EMBED // HEAD_NUM
MODEL_CAT_RATE = 0.02
ID_CAT_RATE = 0.36
IMG_DIM = 4096
TXT_DIM = 1024

_F32 = jnp.float32


def _dot(a, b):
    return jax.lax.dot_general(a, b, (((1,), (0,)), ((), ())),
                               preferred_element_type=_F32)


def _rownorm(x):
    n = jnp.sqrt(jnp.sum(x * x, axis=1, keepdims=True))
    return x / jnp.maximum(n, 1e-12)


def _mhsa_mean(a, b, w_q, w_k, w_cat):
    qa = _dot(a, w_q)
    qb = _dot(b, w_q)
    ka = _dot(a, w_k)
    kb = _dot(b, w_k)
    scale = 1.0 / jnp.sqrt(jnp.float32(D_H))
    z_parts_a = []
    z_parts_b = []
    for h in range(HEAD_NUM):
        s = slice(h * D_H, (h + 1) * D_H)
        qah, qbh = qa[:, s], qb[:, s]
        kah, kbh = ka[:, s], kb[:, s]
        l_aa = jnp.sum(qah * kah, axis=1, keepdims=True) * scale
        l_ab = jnp.sum(qah * kbh, axis=1, keepdims=True) * scale
        l_ba = jnp.sum(qbh * kah, axis=1, keepdims=True) * scale
        l_bb = jnp.sum(qbh * kbh, axis=1, keepdims=True) * scale
        m_a = jnp.maximum(l_aa, l_ab)
        e_aa = jnp.exp(l_aa - m_a)
        e_ab = jnp.exp(l_ab - m_a)
        za = (e_aa * a + e_ab * b) / (e_aa + e_ab)
        m_b = jnp.maximum(l_ba, l_bb)
        e_ba = jnp.exp(l_ba - m_b)
        e_bb = jnp.exp(l_bb - m_b)
        zb = (e_ba * a + e_bb * b) / (e_ba + e_bb)
        z_parts_a.append(za)
        z_parts_b.append(zb)
    zcat_a = jnp.concatenate(z_parts_a, axis=1)
    zcat_b = jnp.concatenate(z_parts_b, axis=1)
    out_a = _dot(zcat_a, w_cat)
    out_b = _dot(zcat_b, w_cat)
    return 0.5 * (out_a + out_b)


def _stage_a_kernel(img_ui_ref, txt_ui_ref, img_iu_ref, txt_iu_ref,
                    img_raw_ref, txt_raw_ref,
                    w_img_ref, b_img_ref, w_txt_ref, b_txt_ref,
                    uemb_full_ref, iemb_full_ref,
                    uemb_blk_ref, iemb_blk_ref,
                    w_q_ref, w_k_ref, w_cat_ref,
                    img_feats_ref, txt_feats_ref,
                    iu_id_ref, tu_id_ref, ug0_ref, ig0_ref):
    img_feats_ref[...] = _dot(img_raw_ref[...], w_img_ref[...]) + b_img_ref[...]
    txt_feats_ref[...] = _dot(txt_raw_ref[...], w_txt_ref[...]) + b_txt_ref[...]
    iemb_full = iemb_full_ref[...]
    uemb_full = uemb_full_ref[...]
    iu_id = _dot(img_ui_ref[...], iemb_full)
    tu_id = _dot(txt_ui_ref[...], iemb_full)
    ii_id = _dot(img_iu_ref[...], uemb_full)
    ti_id = _dot(txt_iu_ref[...], uemb_full)
    iu_id_ref[...] = iu_id
    tu_id_ref[...] = tu_id
    w_q, w_k, w_cat = w_q_ref[...], w_k_ref[...], w_cat_ref[...]
    user_emb = _mhsa_mean(iu_id, tu_id, w_q, w_k, w_cat)
    item_emb = _mhsa_mean(ii_id, ti_id, w_q, w_k, w_cat)
    ug0_ref[...] = uemb_blk_ref[...] + ID_CAT_RATE * _rownorm(user_emb)
    ig0_ref[...] = iemb_blk_ref[...] + ID_CAT_RATE * _rownorm(item_emb)


RM = 256
NBM = N // RM
RE = 1024


def _mega_kernel(ui_ref, iu_ref, rhs_ref, ug0_ref, ig0_ref,
                 iuf_ref, tuf_ref, iif_ref, tif_ref, ufin_ref, ifin_ref,
                 ui16_s, cpack_s, dpack_s, ug2_s):
    p = pl.program_id(0)
    b = pl.program_id(1)
    r0 = b * RM

    @pl.when(p == 0)
    def _phase_c():
        g16 = ui_ref[...].astype(jnp.bfloat16)
        ui16_s[pl.ds(r0, RM), :] = g16
        cp = _dot(g16, rhs_ref[...])
        cpack_s[pl.ds(r0, RM), :] = cp.astype(jnp.bfloat16)
        iuf_ref[...] = cp[:, 0:EMBED]
        tuf_ref[...] = cp[:, EMBED:2 * EMBED]

    @pl.when(p == 1)
    def _phase_d():
        g16 = iu_ref[...].astype(jnp.bfloat16)
        dp = _dot(g16, cpack_s[...])
        dpack_s[pl.ds(r0, RM), :] = dp.astype(jnp.bfloat16)
        iif_ref[...] = dp[:, 0:EMBED]
        tif_ref[...] = dp[:, EMBED:2 * EMBED]

    @pl.when((p == 2) & (b < N // RE))
    def _phase_e():
        e0 = b * RE
        g16 = ui16_s[pl.ds(e0, RE), :]
        t = _dot(g16, dpack_s[:, 2 * EMBED:3 * EMBED])
        ug2 = jax.nn.softmax(t, axis=-1)
        ug2_s[pl.ds(e0, RE), :] = ug2.astype(jnp.bfloat16)
        cp = cpack_s[pl.ds(e0, RE), :].astype(_F32)
        iuf = cp[:, 0:EMBED]
        tuf = cp[:, EMBED:2 * EMBED]
        ug1 = cp[:, 2 * EMBED:3 * EMBED]
        u = (ug0_ref[pl.ds(e0, RE), :] + ug1 + ug2) * (1.0 / 3.0)
        ufin_ref[pl.ds(e0, RE), :] = (u + MODEL_CAT_RATE * _rownorm(iuf)
                                      + MODEL_CAT_RATE * _rownorm(tuf))

    @pl.when(p == 3)
    def _phase_f():
        g16 = iu_ref[...].astype(jnp.bfloat16)
        t = _dot(g16, ug2_s[...])
        ig2 = jax.nn.softmax(t, axis=-1)
        dp = dpack_s[pl.ds(r0, RM), :].astype(_F32)
        iif = dp[:, 0:EMBED]
        tif = dp[:, EMBED:2 * EMBED]
        ig1 = dp[:, 2 * EMBED:3 * EMBED]
        i = (ig0_ref[...] + ig1 + ig2) * (1.0 / 3.0)
        ifin_ref[...] = (i + MODEL_CAT_RATE * _rownorm(iif)
                         + MODEL_CAT_RATE * _rownorm(tif))


def _row_spec(r, cols):
    return pl.BlockSpec((r, cols), lambda b: (b, 0))


def _full_spec(rows, cols):
    return pl.BlockSpec((rows, cols), lambda b: (0, 0))


_ARB = pltpu.CompilerParams(dimension_semantics=("arbitrary",))


def kernel(ui_graph, iu_graph, image_ui_graph, image_iu_graph, text_ui_graph,
           text_iu_graph, image_feats_raw, text_feats_raw, W_img, b_img,
           W_txt, b_txt, user_id_emb, item_id_emb, w_q, w_k, w_cat):
    f32 = _F32
    b_img2 = b_img.reshape(1, EMBED)
    b_txt2 = b_txt.reshape(1, EMBED)

    RA = 256
    (image_feats, text_feats, image_user_id, text_user_id,
     u_g0, i_g0) = pl.pallas_call(
        _stage_a_kernel,
        grid=(N // RA,),
        in_specs=[
            _row_spec(RA, N),
            _row_spec(RA, N),
            _row_spec(RA, N),
            _row_spec(RA, N),
            _row_spec(RA, IMG_DIM),
            _row_spec(RA, TXT_DIM),
            _full_spec(IMG_DIM, EMBED),
            _full_spec(1, EMBED),
            _full_spec(TXT_DIM, EMBED),
            _full_spec(1, EMBED),
            _full_spec(N, EMBED),
            _full_spec(N, EMBED),
            _row_spec(RA, EMBED),
            _row_spec(RA, EMBED),
            _full_spec(EMBED, EMBED),
            _full_spec(EMBED, EMBED),
            _full_spec(HEAD_NUM * EMBED, EMBED),
        ],
        out_specs=[_row_spec(RA, EMBED)] * 6,
        out_shape=[jax.ShapeDtypeStruct((N, EMBED), f32)] * 6,
        compiler_params=_ARB,
    )(image_ui_graph, text_ui_graph, image_iu_graph, text_iu_graph,
      image_feats_raw, text_feats_raw, W_img, b_img2, W_txt, b_txt2,
      user_id_emb, item_id_emb, user_id_emb, item_id_emb, w_q, w_k, w_cat)

    rhs_c = jnp.concatenate([image_feats, text_feats, i_g0],
                            axis=1).astype(jnp.bfloat16)
    last = NBM - 1

    (image_user_feats, text_user_feats, image_item_feats, text_item_feats,
     u_g, i_g) = pl.pallas_call(
        _mega_kernel,
        grid=(4, NBM),
        in_specs=[
            pl.BlockSpec((RM, N),
                         lambda p, b: (jnp.where(p == 0, b, last), 0)),
            pl.BlockSpec((RM, N),
                         lambda p, b: (jnp.where((p == 1) | (p == 3), b, 0), 0)),
            pl.BlockSpec((N, 3 * EMBED), lambda p, b: (0, 0)),
            pl.BlockSpec((N, EMBED), lambda p, b: (0, 0)),
            pl.BlockSpec((RM, EMBED),
                         lambda p, b: (jnp.where(p == 3, b, 0), 0)),
        ],
        out_specs=[
            pl.BlockSpec((RM, EMBED),
                         lambda p, b: (jnp.where(p == 0, b, last), 0)),
            pl.BlockSpec((RM, EMBED),
                         lambda p, b: (jnp.where(p == 0, b, last), 0)),
            pl.BlockSpec((RM, EMBED),
                         lambda p, b: (jnp.where(p == 1, b,
                                       jnp.where(p < 1, 0, last)), 0)),
            pl.BlockSpec((RM, EMBED),
                         lambda p, b: (jnp.where(p == 1, b,
                                       jnp.where(p < 1, 0, last)), 0)),
            pl.BlockSpec((N, EMBED), lambda p, b: (0, 0)),
            pl.BlockSpec((RM, EMBED),
                         lambda p, b: (jnp.where(p == 3, b, 0), 0)),
        ],
        out_shape=[jax.ShapeDtypeStruct((N, EMBED), f32)] * 6,
        scratch_shapes=[
            pltpu.VMEM((N, N), jnp.bfloat16),
            pltpu.VMEM((N, 3 * EMBED), jnp.bfloat16),
            pltpu.VMEM((N, 3 * EMBED), jnp.bfloat16),
            pltpu.VMEM((N, EMBED), jnp.bfloat16),
        ],
        compiler_params=pltpu.CompilerParams(
            dimension_semantics=("arbitrary", "arbitrary"),
            vmem_limit_bytes=64 * 1024 * 1024),
    )(ui_graph, iu_graph, rhs_c, u_g0, i_g0)

    return (u_g, i_g, image_item_feats, text_item_feats, image_user_feats,
            text_user_feats, u_g, i_g, image_user_id, text_user_id)
